# baseline (speedup 1.0000x reference)
import functools

import jax
import jax.numpy as jnp
from jax import lax
from jax.experimental import pallas as pl
from jax.experimental.pallas import tpu as pltpu
from jax.experimental.pallas import tpu_sc as plsc

N = 10000
E = 320000
NC = 2
NS = 16
NW = NC * NS
E_PER_TILE = E // NW
PADE = 240
EPT_P = E_PER_TILE + PADE
KD = 128
NCHD = EPT_P // KD
K = 40
GK = 8
NCH = EPT_P // K
NG = NCH // GK
N_PAD = 10240
ROWS_PER_TILE = N_PAD // NS

MESH = plsc.VectorSubcoreMesh(
    core_axis_name="c", subcore_axis_name="s", num_cores=NC, num_subcores=NS)


def _sc_deg(ei_p, zero_deg):
    @functools.partial(
        pl.kernel,
        out_type=jax.ShapeDtypeStruct((NW, N_PAD), jnp.float32),
        mesh=MESH,
        scratch_types=[
            pltpu.VMEM((NCHD, KD), jnp.int32),
            pltpu.VMEM((N_PAD,), jnp.float32),
        ],
        compiler_params=pltpu.CompilerParams(needs_layout_passes=False))
    def deg_kernel(ei_hbm, zdeg_hbm, deg_out, dst_v, deg_v):
        c = lax.axis_index("c")
        s = lax.axis_index("s")
        w = c * NS + s
        pltpu.sync_copy(ei_hbm.at[1, w], dst_v)
        pltpu.sync_copy(zdeg_hbm, deg_v)
        ones16 = jnp.full((16,), 1.0, jnp.float32)

        def body(j, carry):
            for q in range(KD // 16):
                plsc.addupdate_scatter(
                    deg_v, [dst_v[j, pl.ds(q * 16, 16)]], ones16)
            return carry

        lax.fori_loop(0, NCHD, body, 0)
        pltpu.sync_copy(deg_v, deg_out.at[w])

    return deg_kernel(ei_p, zero_deg)


def _sc_agg(z, ei_p, zero_rows):
    D = z.shape[1]

    @functools.partial(
        pl.kernel,
        out_type=jax.ShapeDtypeStruct((NC, N_PAD, D), jnp.float32),
        mesh=MESH,
        scratch_types=[
            pltpu.VMEM((2 * GK, K), jnp.int32),
            pltpu.VMEM((2 * GK, K), jnp.int32),
            pltpu.VMEM((GK, K, D), jnp.float32),
            pltpu.VMEM_SHARED((N_PAD, D), jnp.float32),
            pltpu.SemaphoreType.DMA,
        ] + [pltpu.SemaphoreType.DMA] * (2 * GK),
        compiler_params=pltpu.CompilerParams(needs_layout_passes=False))
    def agg(z_hbm, ei_hbm, zrow_hbm, acc_out, src_i, dst_i, rows_v, acc_sh,
            isem, *sems):
        gsem, ssem = sems[:GK], sems[GK:]
        c = lax.axis_index("c")
        s = lax.axis_index("s")
        w = c * NS + s

        pltpu.async_copy(ei_hbm.at[0, w, 0], src_i.at[pl.ds(0, GK)], isem)
        pltpu.async_copy(ei_hbm.at[1, w, 0], dst_i.at[pl.ds(0, GK)], isem)
        r0 = s * ROWS_PER_TILE
        pltpu.sync_copy(zrow_hbm, acc_sh.at[pl.ds(r0, ROWS_PER_TILE)])
        plsc.subcore_barrier()

        def run_group(g, p):
            pltpu.make_async_copy(
                ei_hbm.at[0, w, g], src_i.at[pl.ds(p * GK, GK)], isem).wait()
            pltpu.make_async_copy(
                ei_hbm.at[1, w, g], dst_i.at[pl.ds(p * GK, GK)], isem).wait()

            @pl.when(g < NG - 1)
            def _():
                pltpu.async_copy(ei_hbm.at[0, w, g + 1],
                                 src_i.at[pl.ds((1 - p) * GK, GK)], isem)
                pltpu.async_copy(ei_hbm.at[1, w, g + 1],
                                 dst_i.at[pl.ds((1 - p) * GK, GK)], isem)

            gathers = []
            for u in range(GK):
                gathers.append(pltpu.async_copy(
                    z_hbm.at[src_i.at[p * GK + u]], rows_v.at[u], gsem[u]))
            scatters = []
            for u in range(GK):
                gathers[u].wait()
                scatters.append(pltpu.async_copy(
                    rows_v.at[u], acc_sh.at[dst_i.at[p * GK + u]], ssem[u],
                    add=True))
            for u in range(GK):
                scatters[u].wait()

        def body(gg, carry):
            run_group(2 * gg, 0)
            run_group(2 * gg + 1, 1)
            return carry

        lax.fori_loop(0, NG // 2, body, 0)
        plsc.subcore_barrier()

        pltpu.sync_copy(acc_sh.at[pl.ds(r0, ROWS_PER_TILE)],
                        acc_out.at[c, pl.ds(r0, ROWS_PER_TILE)])

    return agg(z, ei_p, zero_rows)


ROW_BLK = 1024
GRID = N_PAD // ROW_BLK


def _deg_col(deg_blk):
    ones = jnp.ones((NW, 1), jnp.float32)
    d = lax.dot_general(deg_blk, ones, (((0,), (0,)), ((), ())),
                        preferred_element_type=jnp.float32)
    return jnp.maximum(d, 1.0)


def _tc_stage0(h, W_self0, W_neigh0, b0):
    def body(h_ref, ws_ref, wn_ref, b_ref, s_ref, z_ref):
        hblk = h_ref[...]
        s_ref[...] = jnp.dot(hblk, ws_ref[...],
                             preferred_element_type=jnp.float32) + b_ref[...]
        z_ref[...] = jnp.dot(hblk, wn_ref[...],
                             preferred_element_type=jnp.float32)

    H = W_self0.shape[1]
    return pl.pallas_call(
        body,
        grid=(GRID,),
        in_specs=[
            pl.BlockSpec((ROW_BLK, h.shape[1]), lambda i: (i, 0)),
            pl.BlockSpec(W_self0.shape, lambda i: (0, 0)),
            pl.BlockSpec(W_neigh0.shape, lambda i: (0, 0)),
            pl.BlockSpec((1, H), lambda i: (0, 0)),
        ],
        out_specs=[
            pl.BlockSpec((ROW_BLK, H), lambda i: (i, 0)),
            pl.BlockSpec((ROW_BLK, H), lambda i: (i, 0)),
        ],
        out_shape=[
            jax.ShapeDtypeStruct((N_PAD, H), jnp.float32),
            jax.ShapeDtypeStruct((N_PAD, H), jnp.float32),
        ],
    )(h, W_self0, W_neigh0, b0)


def _tc_stage1(s0, acc0, deg, W_self1, W_neigh1, b1):
    def body(s0_ref, acc_ref, deg_ref, ws_ref, wn_ref, b_ref, s_ref, z_ref):
        agg = acc_ref[0] + acc_ref[1]
        rdeg = 1.0 / _deg_col(deg_ref[...])
        h1 = jnp.maximum(s0_ref[...] + agg * rdeg, 0.0)
        s_ref[...] = jnp.dot(h1, ws_ref[...],
                             preferred_element_type=jnp.float32) + b_ref[...]
        z_ref[...] = jnp.dot(h1, wn_ref[...],
                             preferred_element_type=jnp.float32)

    H = s0.shape[1]
    C = W_self1.shape[1]
    return pl.pallas_call(
        body,
        grid=(GRID,),
        in_specs=[
            pl.BlockSpec((ROW_BLK, H), lambda i: (i, 0)),
            pl.BlockSpec((NC, ROW_BLK, H), lambda i: (0, i, 0)),
            pl.BlockSpec((NW, ROW_BLK), lambda i: (0, i)),
            pl.BlockSpec(W_self1.shape, lambda i: (0, 0)),
            pl.BlockSpec(W_neigh1.shape, lambda i: (0, 0)),
            pl.BlockSpec((1, C), lambda i: (0, 0)),
        ],
        out_specs=[
            pl.BlockSpec((ROW_BLK, C), lambda i: (i, 0)),
            pl.BlockSpec((ROW_BLK, C), lambda i: (i, 0)),
        ],
        out_shape=[
            jax.ShapeDtypeStruct((N_PAD, C), jnp.float32),
            jax.ShapeDtypeStruct((N_PAD, C), jnp.float32),
        ],
    )(s0, acc0, deg, W_self1, W_neigh1, b1)


def _tc_stage2(s1, acc1, deg, C):
    def body(s1_ref, acc_ref, deg_ref, o_ref):
        agg = acc_ref[0] + acc_ref[1]
        rdeg = 1.0 / _deg_col(deg_ref[...])
        o_ref[...] = (s1_ref[...] + agg * rdeg)[:, :C]

    H = s1.shape[1]
    return pl.pallas_call(
        body,
        grid=(GRID,),
        in_specs=[
            pl.BlockSpec((ROW_BLK, H), lambda i: (i, 0)),
            pl.BlockSpec((NC, ROW_BLK, H), lambda i: (0, i, 0)),
            pl.BlockSpec((NW, ROW_BLK), lambda i: (0, i)),
        ],
        out_specs=pl.BlockSpec((ROW_BLK, C), lambda i: (i, 0)),
        out_shape=jax.ShapeDtypeStruct((N, C), jnp.float32),
    )(s1, acc1, deg)


def kernel(h, edge_index, W_self0, W_neigh0, b0, W_self1, W_neigh1, b1):
    ei_t = edge_index.reshape(2, NW, E_PER_TILE)
    paddst = jnp.broadcast_to(
        N + jnp.arange(PADE, dtype=jnp.int32), (1, NW, PADE))
    padblk = jnp.concatenate(
        [jnp.zeros((1, NW, PADE), jnp.int32), paddst], axis=0)
    ei_pad = jnp.concatenate([ei_t, padblk], axis=2)
    ei_p = ei_pad.reshape(2, NW, NG, GK, K)
    ei_d = ei_pad.reshape(2, NW, NCHD, KD)

    zero128 = jnp.zeros((ROWS_PER_TILE, 128), jnp.float32)
    zero_deg = jnp.zeros((N_PAD,), jnp.float32)

    C = W_self1.shape[1]
    pad = ((0, 0), (0, 128 - C))
    Ws1 = jnp.pad(W_self1, pad)
    Wn1 = jnp.pad(W_neigh1, pad)
    b1p = jnp.pad(b1, ((0, 128 - C),))

    deg = _sc_deg(ei_d, zero_deg)
    s0, z0 = _tc_stage0(h, W_self0, W_neigh0, b0.reshape(1, -1))
    acc0 = _sc_agg(z0, ei_p, zero128)
    s1, z1 = _tc_stage1(s0, acc0, deg, Ws1, Wn1, b1p.reshape(1, -1))
    acc1 = _sc_agg(z1, ei_p, zero128)
    return _tc_stage2(s1, acc1, deg, C)


# speedup vs baseline: 2.4517x; 2.4517x over previous
import functools

import jax
import jax.numpy as jnp
from jax import lax
from jax.experimental import pallas as pl
from jax.experimental.pallas import tpu as pltpu
from jax.experimental.pallas import tpu_sc as plsc

N = 10000
E = 320000
NC = 2
NS = 16
NW = NC * NS
E_PER_CORE = E // NC
E_PER_TILE = E // NW
K = 40
NCHUNK = E_PER_TILE // K
GK = 5
NG = NCHUNK // GK
N_PAD = 10240
ROWS_PER_TILE = N_PAD // NS


def _sc_agg(z, ei4, zero_rows, zero_deg, with_deg):
    D = z.shape[1]
    mesh = plsc.VectorSubcoreMesh(
        core_axis_name="c", subcore_axis_name="s", num_cores=NC,
        num_subcores=NS)

    out_type = [jax.ShapeDtypeStruct((NC, N_PAD, D), jnp.float32)]
    scratch = [
        pltpu.VMEM((2 * GK, K), jnp.int32),
        pltpu.VMEM((2 * GK, K), jnp.int32),
        pltpu.VMEM((GK, K, D), jnp.float32),
        pltpu.VMEM_SHARED((N_PAD, D), jnp.float32),
        pltpu.SemaphoreType.DMA,
    ] + [pltpu.SemaphoreType.DMA] * (2 * GK)
    if with_deg:
        out_type.append(jax.ShapeDtypeStruct((NW, N_PAD), jnp.float32))
        scratch.append(pltpu.VMEM((N_PAD,), jnp.float32))

    @functools.partial(
        pl.kernel, out_type=out_type, mesh=mesh, scratch_types=scratch,
        compiler_params=pltpu.CompilerParams(needs_layout_passes=False))
    def agg(z_hbm, ei_hbm, zrow_hbm, zdeg_hbm, *rest):
        if with_deg:
            (acc_out, deg_out, src_i, dst_i, rows_v, acc_sh, isem, *sems,
             deg_v) = rest
        else:
            acc_out, src_i, dst_i, rows_v, acc_sh, isem, *sems = rest
        gsem, ssem = sems[:GK], sems[GK:]
        c = lax.axis_index("c")
        s = lax.axis_index("s")
        w = c * NS + s

        pltpu.async_copy(ei_hbm.at[0, w, 0], src_i.at[pl.ds(0, GK)], isem)
        pltpu.async_copy(ei_hbm.at[1, w, 0], dst_i.at[pl.ds(0, GK)], isem)
        r0 = s * ROWS_PER_TILE
        pltpu.sync_copy(zrow_hbm, acc_sh.at[pl.ds(r0, ROWS_PER_TILE)])
        if with_deg:
            pltpu.sync_copy(zdeg_hbm, deg_v)
            ones16 = jnp.full((16,), 1.0, jnp.float32)
            tailmask = lax.iota(jnp.int32, 16) >= 8
        plsc.subcore_barrier()

        def run_group(g, p):
            pltpu.make_async_copy(
                ei_hbm.at[0, w, g], src_i.at[pl.ds(p * GK, GK)], isem).wait()
            pltpu.make_async_copy(
                ei_hbm.at[1, w, g], dst_i.at[pl.ds(p * GK, GK)], isem).wait()

            @pl.when(g < NG - 1)
            def _():
                pltpu.async_copy(ei_hbm.at[0, w, g + 1],
                                 src_i.at[pl.ds((1 - p) * GK, GK)], isem)
                pltpu.async_copy(ei_hbm.at[1, w, g + 1],
                                 dst_i.at[pl.ds((1 - p) * GK, GK)], isem)

            gathers = []
            for u in range(GK):
                gathers.append(pltpu.async_copy(
                    z_hbm.at[src_i.at[p * GK + u]], rows_v.at[u], gsem[u]))
            if with_deg:
                for u in range(GK):
                    r = p * GK + u
                    plsc.addupdate_scatter(
                        deg_v, [dst_i[r, pl.ds(0, 16)]], ones16)
                    plsc.addupdate_scatter(
                        deg_v, [dst_i[r, pl.ds(16, 16)]], ones16)
                    plsc.addupdate_scatter(
                        deg_v, [dst_i[r, pl.ds(24, 16)]], ones16,
                        mask=tailmask)
            scatters = []
            for u in range(GK):
                gathers[u].wait()
                scatters.append(pltpu.async_copy(
                    rows_v.at[u], acc_sh.at[dst_i.at[p * GK + u]], ssem[u],
                    add=True))
            for u in range(GK):
                scatters[u].wait()

        def body(gg, carry):
            run_group(2 * gg, 0)
            run_group(2 * gg + 1, 1)
            return carry

        lax.fori_loop(0, NG // 2, body, 0)
        plsc.subcore_barrier()

        pltpu.sync_copy(acc_sh.at[pl.ds(r0, ROWS_PER_TILE)],
                        acc_out.at[c, pl.ds(r0, ROWS_PER_TILE)])
        if with_deg:
            pltpu.sync_copy(deg_v, deg_out.at[w])

    res = agg(z, ei4, zero_rows, zero_deg)
    if not isinstance(res, (list, tuple)):
        res = (res,)
    return res[0] if not with_deg else tuple(res)


ROW_BLK = 1024
GRID = N_PAD // ROW_BLK


def _deg_col(deg_blk):
    ones = jnp.ones((NW, 1), jnp.float32)
    d = lax.dot_general(deg_blk, ones, (((0,), (0,)), ((), ())),
                        preferred_element_type=jnp.float32)
    return jnp.maximum(d, 1.0)


def _tc_stage0(h, W_self0, W_neigh0, b0):
    def body(h_ref, ws_ref, wn_ref, b_ref, s_ref, z_ref):
        hblk = h_ref[...]
        s_ref[...] = jnp.dot(hblk, ws_ref[...],
                             preferred_element_type=jnp.float32) + b_ref[...]
        z_ref[...] = jnp.dot(hblk, wn_ref[...],
                             preferred_element_type=jnp.float32)

    H = W_self0.shape[1]
    return pl.pallas_call(
        body,
        grid=(GRID,),
        in_specs=[
            pl.BlockSpec((ROW_BLK, h.shape[1]), lambda i: (i, 0)),
            pl.BlockSpec(W_self0.shape, lambda i: (0, 0)),
            pl.BlockSpec(W_neigh0.shape, lambda i: (0, 0)),
            pl.BlockSpec((1, H), lambda i: (0, 0)),
        ],
        out_specs=[
            pl.BlockSpec((ROW_BLK, H), lambda i: (i, 0)),
            pl.BlockSpec((ROW_BLK, H), lambda i: (i, 0)),
        ],
        out_shape=[
            jax.ShapeDtypeStruct((N_PAD, H), jnp.float32),
            jax.ShapeDtypeStruct((N_PAD, H), jnp.float32),
        ],
    )(h, W_self0, W_neigh0, b0)


def _tc_stage1(s0, acc0, deg, W_self1, W_neigh1, b1):
    def body(s0_ref, acc_ref, deg_ref, ws_ref, wn_ref, b_ref, s_ref, z_ref):
        agg = acc_ref[0] + acc_ref[1]
        rdeg = 1.0 / _deg_col(deg_ref[...])
        h1 = jnp.maximum(s0_ref[...] + agg * rdeg, 0.0)
        s_ref[...] = jnp.dot(h1, ws_ref[...],
                             preferred_element_type=jnp.float32) + b_ref[...]
        z_ref[...] = jnp.dot(h1, wn_ref[...],
                             preferred_element_type=jnp.float32)

    H = s0.shape[1]
    C = W_self1.shape[1]
    return pl.pallas_call(
        body,
        grid=(GRID,),
        in_specs=[
            pl.BlockSpec((ROW_BLK, H), lambda i: (i, 0)),
            pl.BlockSpec((NC, ROW_BLK, H), lambda i: (0, i, 0)),
            pl.BlockSpec((NW, ROW_BLK), lambda i: (0, i)),
            pl.BlockSpec(W_self1.shape, lambda i: (0, 0)),
            pl.BlockSpec(W_neigh1.shape, lambda i: (0, 0)),
            pl.BlockSpec((1, C), lambda i: (0, 0)),
        ],
        out_specs=[
            pl.BlockSpec((ROW_BLK, C), lambda i: (i, 0)),
            pl.BlockSpec((ROW_BLK, C), lambda i: (i, 0)),
        ],
        out_shape=[
            jax.ShapeDtypeStruct((N_PAD, C), jnp.float32),
            jax.ShapeDtypeStruct((N_PAD, C), jnp.float32),
        ],
    )(s0, acc0, deg, W_self1, W_neigh1, b1)


def _tc_stage2(s1, acc1, deg, C):
    def body(s1_ref, acc_ref, deg_ref, o_ref):
        agg = acc_ref[0] + acc_ref[1]
        rdeg = 1.0 / _deg_col(deg_ref[...])
        o_ref[...] = (s1_ref[...] + agg * rdeg)[:, :C]

    H = s1.shape[1]
    return pl.pallas_call(
        body,
        grid=(GRID,),
        in_specs=[
            pl.BlockSpec((ROW_BLK, H), lambda i: (i, 0)),
            pl.BlockSpec((NC, ROW_BLK, H), lambda i: (0, i, 0)),
            pl.BlockSpec((NW, ROW_BLK), lambda i: (0, i)),
        ],
        out_specs=pl.BlockSpec((ROW_BLK, C), lambda i: (i, 0)),
        out_shape=jax.ShapeDtypeStruct((N, C), jnp.float32),
    )(s1, acc1, deg)


def kernel(h, edge_index, W_self0, W_neigh0, b0, W_self1, W_neigh1, b1):
    ei4 = edge_index.reshape(2, NW, NG, GK, K)
    zero128 = jnp.zeros((ROWS_PER_TILE, 128), jnp.float32)
    zero_deg = jnp.zeros((N_PAD,), jnp.float32)

    C = W_self1.shape[1]
    pad = ((0, 0), (0, 128 - C))
    Ws1 = jnp.pad(W_self1, pad)
    Wn1 = jnp.pad(W_neigh1, pad)
    b1p = jnp.pad(b1, ((0, 128 - C),))

    s0, z0 = _tc_stage0(h, W_self0, W_neigh0, b0.reshape(1, -1))
    acc0, deg = _sc_agg(z0, ei4, zero128, zero_deg, with_deg=True)
    s1, z1 = _tc_stage1(s0, acc0, deg, Ws1, Wn1, b1p.reshape(1, -1))
    acc1 = _sc_agg(z1, ei4, zero128, zero_deg, with_deg=False)
    return _tc_stage2(s1, acc1, deg, C)
